# Initial kernel scaffold; baseline (speedup 1.0000x reference)
import functools
import math

import jax
import jax.numpy as jnp
from jax import lax
from jax.experimental import pallas as pl
from jax.experimental.pallas import tpu as pltpu
from jax.experimental.pallas import tpu_sc as plsc

N = 4096
T = 12
IN_FEAT = 16
W_DIM = 8
D = 64
KD = 32
TOPK = 16
N_ADJ = 65536

FH = (T // 2) * D
N_ENT = N * TOPK + N_ADJ
EPT = N_ENT // 16
CHUNK = 64
NCH = EPT // CHUNK


def _full_spec(shape):
    nd = len(shape)
    return pl.BlockSpec(shape, lambda i, _nd=nd: (0,) * _nd)


def _mlp_body(xe_ref, xw_ref, ew1, eb1, ew2, eb2, ww1, wb1, ww2, wb2, cw, cb,
              h0_ref):
    xe = xe_ref[...]
    he = jnp.maximum(xe @ ew1[...] + eb1[...], 0.0) @ ew2[...] + eb2[...]
    xw = xw_ref[...]
    hw = jnp.maximum(xw @ ww1[...] + wb1[...], 0.0) @ ww2[...] + wb2[...]
    hc = jnp.concatenate([he, hw], axis=1)
    h0_ref[...] = jnp.maximum(hc @ cw[...] + cb[...], 0.0)


def _mlp_call(xe, xw, p):
    R = 6144
    grid = (N * T) // R
    return pl.pallas_call(
        _mlp_body,
        grid=(grid,),
        in_specs=[
            pl.BlockSpec((R, IN_FEAT), lambda i: (i, 0)),
            pl.BlockSpec((R, W_DIM), lambda i: (i, 0)),
            _full_spec((IN_FEAT, 32)), _full_spec((1, 32)),
            _full_spec((32, D)), _full_spec((1, D)),
            _full_spec((W_DIM, 32)), _full_spec((1, 32)),
            _full_spec((32, D)), _full_spec((1, D)),
            _full_spec((2 * D, D)), _full_spec((1, D)),
        ],
        out_specs=pl.BlockSpec((R, D), lambda i: (i, 0)),
        out_shape=jax.ShapeDtypeStruct((N * T, D), jnp.float32),
    )(xe, xw,
      p['e_W1'], p['e_b1'].reshape(1, 32), p['e_W2'], p['e_b2'].reshape(1, D),
      p['w_W1'], p['w_b1'].reshape(1, 32), p['w_W2'], p['w_b2'].reshape(1, D),
      p['c_W'], p['c_b'].reshape(1, D))


def _pool_body(h0_ref, wq, wk, q_ref, k_ref):
    hm = jnp.mean(h0_ref[...], axis=1)
    q_ref[...] = hm @ wq[...]
    k_ref[...] = hm @ wk[...]


def _pool_call(h0, wq, wk):
    R = 512
    return pl.pallas_call(
        _pool_body,
        grid=(N // R,),
        in_specs=[
            pl.BlockSpec((R, T, D), lambda i: (i, 0, 0)),
            _full_spec((D, KD)), _full_spec((D, KD)),
        ],
        out_specs=[
            pl.BlockSpec((R, KD), lambda i: (i, 0)),
            pl.BlockSpec((R, KD), lambda i: (i, 0)),
        ],
        out_shape=[
            jax.ShapeDtypeStruct((N, KD), jnp.float32),
            jax.ShapeDtypeStruct((N, KD), jnp.float32),
        ],
    )(h0, wq, wk)


def _topk_body(gm_ref, q_ref, k_ref, idx_ref, w_ref):
    s = lax.dot_general(q_ref[...], k_ref[...], (((1,), (1,)), ((), ())),
                        preferred_element_type=jnp.float32)
    s = s * (1.0 / math.sqrt(KD))
    cols = lax.broadcasted_iota(jnp.int32, s.shape, 1)
    neg = jnp.float32(-3e38)
    vals = []
    idxs = []
    for _ in range(TOPK):
        m = jnp.max(s, axis=1, keepdims=True)
        am = jnp.min(jnp.where(s == m, cols, jnp.int32(N)), axis=1,
                     keepdims=True)
        vals.append(m)
        idxs.append(am)
        s = jnp.where(cols == am, neg, s)
    v = jnp.concatenate(vals, axis=1)
    mi = jnp.concatenate(idxs, axis=1)
    e = jnp.exp(v - v[:, :1])
    w = e / jnp.sum(e, axis=1, keepdims=True)
    idx_ref[...] = mi
    w_ref[...] = w * gm_ref[0, 0]


def _topk_call(q, k, gm):
    R = 256
    return pl.pallas_call(
        _topk_body,
        grid=(N // R,),
        in_specs=[
            pl.BlockSpec(memory_space=pltpu.SMEM),
            pl.BlockSpec((R, KD), lambda i: (i, 0)),
            _full_spec((N, KD)),
        ],
        out_specs=[
            pl.BlockSpec((R, TOPK), lambda i: (i, 0)),
            pl.BlockSpec((R, TOPK), lambda i: (i, 0)),
        ],
        out_shape=[
            jax.ShapeDtypeStruct((N, TOPK), jnp.int32),
            jax.ShapeDtypeStruct((N, TOPK), jnp.float32),
        ],
    )(gm, q, k)


def _svals_body(rows_h, cols_h, vals_h, g_h, svals_h,
                hist, degl, tmp, dinv, rbuf, cbuf, vbuf, svbuf, gbuf,
                deg_sh, dinv_sh):
    c = lax.axis_index("c")
    s = lax.axis_index("s")
    zero16 = jnp.zeros((16,), jnp.float32)
    lanes = lax.iota(jnp.int32, 16)

    def zhist(i, _):
        hist[pl.ds(i * 16, 16)] = zero16
        return 0
    lax.fori_loop(0, N, zhist, 0)

    base1 = s * (N_ADJ // 16)
    pltpu.sync_copy(rows_h.at[pl.ds(base1, N_ADJ // 16)], rbuf)
    pltpu.sync_copy(vals_h.at[pl.ds(base1, N_ADJ // 16)], vbuf)

    def hbody(i, _):
        rv = rbuf[pl.ds(i * 16, 16)]
        vv = vbuf[pl.ds(i * 16, 16)]
        plsc.addupdate_scatter(hist, [lanes * N + rv], vv)
        return 0
    lax.fori_loop(0, (N_ADJ // 16) // 16, hbody, 0)

    def rbody(j, _):
        acc = hist[pl.ds(j * 16, 16)]
        for l in range(1, 16):
            acc = acc + hist[pl.ds(l * N + j * 16, 16)]
        degl[pl.ds(j * 16, 16)] = acc
        return 0
    lax.fori_loop(0, N // 16, rbody, 0)

    pltpu.sync_copy(degl, deg_sh.at[s])
    plsc.subcore_barrier()
    stripe = s * (N // 16)

    def sall(l, _):
        @pl.when(l != s)
        def _():
            pltpu.sync_copy(deg_sh.at[l, pl.ds(stripe, N // 16)],
                            tmp.at[pl.ds(0, N // 16)])

            def addc(j, __):
                degl[pl.ds(stripe + j * 16, 16)] = (
                    degl[pl.ds(stripe + j * 16, 16)]
                    + tmp[pl.ds(j * 16, 16)])
                return 0
            lax.fori_loop(0, (N // 16) // 16, addc, 0)
        return 0
    lax.fori_loop(0, 16, sall, 0)

    def nbody(j, _):
        x = degl[pl.ds(stripe + j * 16, 16)]
        xs = x + jnp.float32(1e-8)
        i0 = plsc.bitcast(xs, jnp.int32)
        y = plsc.bitcast(
            jnp.int32(0x5F3759DF) - lax.shift_right_logical(i0, 1),
            jnp.float32)
        for _i in range(3):
            y = y * (jnp.float32(1.5) - jnp.float32(0.5) * xs * y * y)
        dinv[pl.ds(j * 16, 16)] = jnp.where(x > 0.0, y, jnp.float32(0.0))
        return 0
    lax.fori_loop(0, (N // 16) // 16, nbody, 0)
    pltpu.sync_copy(dinv.at[pl.ds(0, N // 16)],
                    dinv_sh.at[pl.ds(stripe, N // 16)])
    plsc.subcore_barrier()
    pltpu.sync_copy(dinv_sh, dinv)

    pltpu.sync_copy(g_h, gbuf)
    gv = gbuf[...]
    wid = c * 16 + s
    base2 = wid * (N_ADJ // 32)
    pltpu.sync_copy(rows_h.at[pl.ds(base2, N_ADJ // 32)],
                    rbuf.at[pl.ds(0, N_ADJ // 32)])
    pltpu.sync_copy(cols_h.at[pl.ds(base2, N_ADJ // 32)],
                    cbuf.at[pl.ds(0, N_ADJ // 32)])
    pltpu.sync_copy(vals_h.at[pl.ds(base2, N_ADJ // 32)],
                    vbuf.at[pl.ds(0, N_ADJ // 32)])

    def cbody(i, _):
        rv = rbuf[pl.ds(i * 16, 16)]
        cv = cbuf[pl.ds(i * 16, 16)]
        vv = vbuf[pl.ds(i * 16, 16)]
        dr = plsc.load_gather(dinv, [rv])
        dc = plsc.load_gather(dinv, [cv])
        svbuf[pl.ds(i * 16, 16)] = gv * dr * dc * vv
        return 0
    lax.fori_loop(0, (N_ADJ // 32) // 16, cbody, 0)
    pltpu.sync_copy(svbuf, svals_h.at[pl.ds(base2, N_ADJ // 32)])


_svals_call = functools.partial(
    pl.kernel,
    out_type=jax.ShapeDtypeStruct((N_ADJ,), jnp.float32),
    mesh=plsc.VectorSubcoreMesh(core_axis_name="c", subcore_axis_name="s"),
    scratch_types=[
        pltpu.VMEM((16 * N,), jnp.float32),
        pltpu.VMEM((N,), jnp.float32),
        pltpu.VMEM((N // 16,), jnp.float32),
        pltpu.VMEM((N,), jnp.float32),
        pltpu.VMEM((N_ADJ // 16,), jnp.int32),
        pltpu.VMEM((N_ADJ // 32,), jnp.int32),
        pltpu.VMEM((N_ADJ // 16,), jnp.float32),
        pltpu.VMEM((N_ADJ // 32,), jnp.float32),
        pltpu.VMEM((16,), jnp.float32),
        pltpu.VMEM_SHARED((16, N), jnp.float32),
        pltpu.VMEM_SHARED((N,), jnp.float32),
    ],
)(_svals_body)


def _spmm_body(table_h, rows3_h, cols3_h, coefs_h, out_h,
               acc_sh, rbuf, idxb, cfb, gb0, gb1, gs0, gs1):
    c = lax.axis_index("c")
    s = lax.axis_index("s")
    zero16 = jnp.zeros((16,), jnp.float32)

    pltpu.sync_copy(rows3_h.at[s], rbuf)
    pltpu.sync_copy(cols3_h.at[s], idxb)
    pltpu.sync_copy(coefs_h.at[pl.ds(s * EPT, EPT)], cfb)

    def tr(ch, _):
        for q in range(CHUNK // 16):
            sl = pl.ds(q * 16, 16)
            idxb[ch, sl] = idxb[ch, sl] * 2 + c
        return 0
    lax.fori_loop(0, NCH, tr, 0)

    def zb(r, _):
        for q in range(FH // 16):
            gb0[r, pl.ds(q * 16, 16)] = zero16
        return 0
    lax.fori_loop(0, CHUNK, zb, 0)
    for kk in range(4):
        pltpu.sync_copy(gb0, acc_sh.at[pl.ds(s * 256 + kk * CHUNK, CHUNK)])
    plsc.subcore_barrier()

    def scale(gb, ch):
        def s1(j, _):
            cf = plsc.load_gather(
                cfb, [jnp.full((16,), ch * CHUNK + j, jnp.int32)])
            for q in range(FH // 16):
                sl = pl.ds(q * 16, 16)
                gb[j, sl] = gb[j, sl] * cf
            return 0
        lax.fori_loop(0, CHUNK, s1, 0)

    pltpu.async_copy(table_h.at[idxb.at[0]], gb0, gs0)

    def pair(p_, _):
        ch0 = p_ * 2
        pltpu.make_async_copy(table_h.at[idxb.at[ch0]], gb0, gs0).wait()

        @pl.when(ch0 + 1 < NCH)
        def _():
            pltpu.async_copy(table_h.at[idxb.at[ch0 + 1]], gb1, gs1)
        scale(gb0, ch0)
        pltpu.sync_copy(gb0, acc_sh.at[rbuf.at[ch0]], add=True)

        ch1 = ch0 + 1
        pltpu.make_async_copy(table_h.at[idxb.at[ch1]], gb1, gs1).wait()

        @pl.when(ch1 + 1 < NCH)
        def _():
            pltpu.async_copy(table_h.at[idxb.at[ch1 + 1]], gb0, gs0)
        scale(gb1, ch1)
        pltpu.sync_copy(gb1, acc_sh.at[rbuf.at[ch1]], add=True)
        return 0
    lax.fori_loop(0, NCH // 2, pair, 0)

    plsc.subcore_barrier()
    for kk in range(4):
        row0 = s * 256 + kk * CHUNK
        pltpu.sync_copy(acc_sh.at[pl.ds(row0, CHUNK)],
                        out_h.at[pl.ds(row0, CHUNK), c])


_spmm_call = functools.partial(
    pl.kernel,
    out_type=jax.ShapeDtypeStruct((N, 2, FH), jnp.float32),
    mesh=plsc.VectorSubcoreMesh(core_axis_name="c", subcore_axis_name="s"),
    scratch_types=[
        pltpu.VMEM_SHARED((N, FH), jnp.float32),
        pltpu.VMEM((NCH, CHUNK), jnp.int32),
        pltpu.VMEM((NCH, CHUNK), jnp.int32),
        pltpu.VMEM((EPT,), jnp.float32),
        pltpu.VMEM((CHUNK, FH), jnp.float32),
        pltpu.VMEM((CHUNK, FH), jnp.float32),
        pltpu.SemaphoreType.DMA,
        pltpu.SemaphoreType.DMA,
    ],
)(_spmm_body)


def _block_body(y_ref, hprev_ref, ws, bs, wt, bt, out_ref):
    y = y_ref[...]
    hs = lax.dot_general(y, ws[...], (((2,), (0,)), ((), ())),
                         preferred_element_type=jnp.float32) + bs[...]
    hs = jnp.maximum(hs, 0.0)
    z = jnp.zeros((y.shape[0], 2, D), jnp.float32)
    hp = jnp.concatenate([z, hs], axis=1)
    dn = (((2,), (0,)), ((), ()))
    wt_v = wt[...]
    ht = (lax.dot_general(hp[:, 0:T], wt_v[0], dn,
                          preferred_element_type=jnp.float32)
          + lax.dot_general(hp[:, 1:T + 1], wt_v[1], dn,
                            preferred_element_type=jnp.float32)
          + lax.dot_general(hp[:, 2:T + 2], wt_v[2], dn,
                            preferred_element_type=jnp.float32)
          + bt[...])
    out_ref[...] = jnp.maximum(ht, 0.0) + hprev_ref[...]


def _block_call(y, hprev, ws, bs, wt, bt):
    R = 512
    return pl.pallas_call(
        _block_body,
        grid=(N // R,),
        in_specs=[
            pl.BlockSpec((R, T, D), lambda i: (i, 0, 0)),
            pl.BlockSpec((R, T, D), lambda i: (i, 0, 0)),
            _full_spec((D, D)), _full_spec((1, D)),
            _full_spec((3, D, D)), _full_spec((1, D)),
        ],
        out_specs=pl.BlockSpec((R, T, D), lambda i: (i, 0, 0)),
        out_shape=jax.ShapeDtypeStruct((N, T, D), jnp.float32),
    )(y, hprev, ws, bs.reshape(1, D), wt, bt.reshape(1, D))


def _head_body(h_ref, w1, b1, w2, b2, out_ref):
    hl = h_ref[...][:, T - 1, :]
    x = jnp.maximum(hl @ w1[...] + b1[...], 0.0)
    out_ref[...] = x @ w2[...] + b2[...]


def _head_call(h, w1, b1, w2, b2):
    R = 512
    return pl.pallas_call(
        _head_body,
        grid=(N // R,),
        in_specs=[
            pl.BlockSpec((R, T, D), lambda i: (i, 0, 0)),
            _full_spec((D, D // 2)), _full_spec((1, D // 2)),
            _full_spec((D // 2, 1)), _full_spec((1, 1)),
        ],
        out_specs=pl.BlockSpec((R, 1), lambda i: (i, 0)),
        out_shape=jax.ShapeDtypeStruct((N, 1), jnp.float32),
    )(h, w1, b1.reshape(1, D // 2), w2, b2.reshape(1, 1))


def kernel(X_edges, X_weather_edges, A_s_indices, A_s_values, params):
    p = params
    xe = X_edges.reshape(N * T, IN_FEAT)
    xw = X_weather_edges.reshape(N * T, W_DIM)
    h0 = _mlp_call(xe, xw, p).reshape(N, T, D)
    q, k = _pool_call(h0, p['Wq'], p['Wk'])

    g = jax.nn.sigmoid(p['alpha'])
    gm = (1.0 - g).reshape(1, 1).astype(jnp.float32)
    tidx, tw = _topk_call(q, k, gm)

    rows_s = A_s_indices[0].astype(jnp.int32)
    cols_s = A_s_indices[1].astype(jnp.int32)
    svals = _svals_call(rows_s, cols_s, A_s_values,
                        jnp.full((16,), g, jnp.float32))

    rows_all = jnp.concatenate(
        [jnp.repeat(jnp.arange(N, dtype=jnp.int32), TOPK), rows_s])
    cols_all = jnp.concatenate([tidx.reshape(-1), cols_s])
    coefs_all = jnp.concatenate([tw.reshape(-1), svals])
    rows3 = rows_all.reshape(16, NCH, CHUNK)
    cols3 = cols_all.reshape(16, NCH, CHUNK)

    h = h0
    for pre in ('b0_', 'b1_'):
        y = _spmm_call(h.reshape(2 * N, FH), rows3, cols3, coefs_all)
        h = _block_call(y.reshape(N, T, D), h,
                        p[pre + 'Ws'], p[pre + 'bs'],
                        p[pre + 'Wt'], p[pre + 'bt'])
    return _head_call(h, p['h_W1'], p['h_b1'], p['h_W2'], p['h_b2'])


# speedup vs baseline: 1.7589x; 1.7589x over previous
import functools
import math

import jax
import jax.numpy as jnp
from jax import lax
from jax.experimental import pallas as pl
from jax.experimental.pallas import tpu as pltpu
from jax.experimental.pallas import tpu_sc as plsc

N = 4096
T = 12
IN_FEAT = 16
W_DIM = 8
D = 64
KD = 32
TOPK = 16
N_ADJ = 65536

FH = (T // 2) * D
N_ENT = N * TOPK + N_ADJ
EPT = N_ENT // 16
CHUNK = 64
NCH = EPT // CHUNK


def _full_spec(shape):
    nd = len(shape)
    return pl.BlockSpec(shape, lambda i, _nd=nd: (0,) * _nd)


def _mlp_body(xe_ref, xw_ref, ew1, eb1, ew2, eb2, ww1, wb1, ww2, wb2, cw, cb,
              h0_ref):
    xe = xe_ref[...]
    he = jnp.maximum(xe @ ew1[...] + eb1[...], 0.0) @ ew2[...] + eb2[...]
    xw = xw_ref[...]
    hw = jnp.maximum(xw @ ww1[...] + wb1[...], 0.0) @ ww2[...] + wb2[...]
    hc = jnp.concatenate([he, hw], axis=1)
    h0_ref[...] = jnp.maximum(hc @ cw[...] + cb[...], 0.0)


def _mlp_call(xe, xw, p):
    R = 6144
    grid = (N * T) // R
    return pl.pallas_call(
        _mlp_body,
        grid=(grid,),
        in_specs=[
            pl.BlockSpec((R, IN_FEAT), lambda i: (i, 0)),
            pl.BlockSpec((R, W_DIM), lambda i: (i, 0)),
            _full_spec((IN_FEAT, 32)), _full_spec((1, 32)),
            _full_spec((32, D)), _full_spec((1, D)),
            _full_spec((W_DIM, 32)), _full_spec((1, 32)),
            _full_spec((32, D)), _full_spec((1, D)),
            _full_spec((2 * D, D)), _full_spec((1, D)),
        ],
        out_specs=pl.BlockSpec((R, D), lambda i: (i, 0)),
        out_shape=jax.ShapeDtypeStruct((N * T, D), jnp.float32),
    )(xe, xw,
      p['e_W1'], p['e_b1'].reshape(1, 32), p['e_W2'], p['e_b2'].reshape(1, D),
      p['w_W1'], p['w_b1'].reshape(1, 32), p['w_W2'], p['w_b2'].reshape(1, D),
      p['c_W'], p['c_b'].reshape(1, D))


def _pool_body(h0_ref, wq, wk, q_ref, k_ref):
    hm = jnp.mean(h0_ref[...], axis=1)
    q_ref[...] = hm @ wq[...]
    k_ref[...] = hm @ wk[...]


def _pool_call(h0, wq, wk):
    R = 512
    return pl.pallas_call(
        _pool_body,
        grid=(N // R,),
        in_specs=[
            pl.BlockSpec((R, T, D), lambda i: (i, 0, 0)),
            _full_spec((D, KD)), _full_spec((D, KD)),
        ],
        out_specs=[
            pl.BlockSpec((R, KD), lambda i: (i, 0)),
            pl.BlockSpec((R, KD), lambda i: (i, 0)),
        ],
        out_shape=[
            jax.ShapeDtypeStruct((N, KD), jnp.float32),
            jax.ShapeDtypeStruct((N, KD), jnp.float32),
        ],
    )(h0, wq, wk)


def _topk_body(gm_ref, q_ref, k_ref, idx_ref, w_ref):
    s = lax.dot_general(q_ref[...], k_ref[...], (((1,), (1,)), ((), ())),
                        preferred_element_type=jnp.float32,
                        precision=lax.Precision.HIGHEST)
    s = s * (1.0 / math.sqrt(KD))
    cols = lax.broadcasted_iota(jnp.int32, s.shape, 1)
    neg = jnp.float32(-3e38)
    vals = []
    idxs = []
    for _ in range(TOPK):
        m = jnp.max(s, axis=1, keepdims=True)
        am = jnp.min(jnp.where(s == m, cols, jnp.int32(N)), axis=1,
                     keepdims=True)
        vals.append(m)
        idxs.append(am)
        s = jnp.where(cols == am, neg, s)
    v = jnp.concatenate(vals, axis=1)
    mi = jnp.concatenate(idxs, axis=1)
    e = jnp.exp(v - v[:, :1])
    w = e / jnp.sum(e, axis=1, keepdims=True)
    idx_ref[...] = mi
    w_ref[...] = w * gm_ref[0, 0]


def _topk_call(q, k, gm):
    R = 256
    return pl.pallas_call(
        _topk_body,
        grid=(N // R,),
        in_specs=[
            pl.BlockSpec(memory_space=pltpu.SMEM),
            pl.BlockSpec((R, KD), lambda i: (i, 0)),
            _full_spec((N, KD)),
        ],
        out_specs=[
            pl.BlockSpec((R, TOPK), lambda i: (i, 0)),
            pl.BlockSpec((R, TOPK), lambda i: (i, 0)),
        ],
        out_shape=[
            jax.ShapeDtypeStruct((N, TOPK), jnp.int32),
            jax.ShapeDtypeStruct((N, TOPK), jnp.float32),
        ],
    )(gm, q, k)


def _svals_body(rows_h, cols_h, vals_h, g_h, svals_h,
                hist, degl, tmp, dinv, rbuf, cbuf, vbuf, svbuf, gbuf,
                deg_sh, dinv_sh):
    c = lax.axis_index("c")
    s = lax.axis_index("s")
    zero16 = jnp.zeros((16,), jnp.float32)
    lanes = lax.iota(jnp.int32, 16)

    def zhist(i, _):
        hist[pl.ds(i * 16, 16)] = zero16
        return 0
    lax.fori_loop(0, N, zhist, 0)

    base1 = s * (N_ADJ // 16)
    pltpu.sync_copy(rows_h.at[pl.ds(base1, N_ADJ // 16)], rbuf)
    pltpu.sync_copy(vals_h.at[pl.ds(base1, N_ADJ // 16)], vbuf)

    def hbody(i, _):
        rv = rbuf[pl.ds(i * 16, 16)]
        vv = vbuf[pl.ds(i * 16, 16)]
        plsc.addupdate_scatter(hist, [lanes * N + rv], vv)
        return 0
    lax.fori_loop(0, (N_ADJ // 16) // 16, hbody, 0)

    def rbody(j, _):
        acc = hist[pl.ds(j * 16, 16)]
        for l in range(1, 16):
            acc = acc + hist[pl.ds(l * N + j * 16, 16)]
        degl[pl.ds(j * 16, 16)] = acc
        return 0
    lax.fori_loop(0, N // 16, rbody, 0)

    pltpu.sync_copy(degl, deg_sh.at[s])
    plsc.subcore_barrier()
    stripe = s * (N // 16)

    def sall(l, _):
        @pl.when(l != s)
        def _():
            pltpu.sync_copy(deg_sh.at[l, pl.ds(stripe, N // 16)],
                            tmp.at[pl.ds(0, N // 16)])

            def addc(j, __):
                degl[pl.ds(stripe + j * 16, 16)] = (
                    degl[pl.ds(stripe + j * 16, 16)]
                    + tmp[pl.ds(j * 16, 16)])
                return 0
            lax.fori_loop(0, (N // 16) // 16, addc, 0)
        return 0
    lax.fori_loop(0, 16, sall, 0)

    def nbody(j, _):
        x = degl[pl.ds(stripe + j * 16, 16)]
        xs = x + jnp.float32(1e-8)
        i0 = plsc.bitcast(xs, jnp.int32)
        y = plsc.bitcast(
            jnp.int32(0x5F3759DF) - lax.shift_right_logical(i0, 1),
            jnp.float32)
        for _i in range(3):
            y = y * (jnp.float32(1.5) - jnp.float32(0.5) * xs * y * y)
        dinv[pl.ds(j * 16, 16)] = jnp.where(x > 0.0, y, jnp.float32(0.0))
        return 0
    lax.fori_loop(0, (N // 16) // 16, nbody, 0)
    pltpu.sync_copy(dinv.at[pl.ds(0, N // 16)],
                    dinv_sh.at[pl.ds(stripe, N // 16)])
    plsc.subcore_barrier()
    pltpu.sync_copy(dinv_sh, dinv)

    pltpu.sync_copy(g_h, gbuf)
    gv = gbuf[...]
    wid = c * 16 + s
    base2 = wid * (N_ADJ // 32)
    pltpu.sync_copy(rows_h.at[pl.ds(base2, N_ADJ // 32)],
                    rbuf.at[pl.ds(0, N_ADJ // 32)])
    pltpu.sync_copy(cols_h.at[pl.ds(base2, N_ADJ // 32)],
                    cbuf.at[pl.ds(0, N_ADJ // 32)])
    pltpu.sync_copy(vals_h.at[pl.ds(base2, N_ADJ // 32)],
                    vbuf.at[pl.ds(0, N_ADJ // 32)])

    def cbody(i, _):
        rv = rbuf[pl.ds(i * 16, 16)]
        cv = cbuf[pl.ds(i * 16, 16)]
        vv = vbuf[pl.ds(i * 16, 16)]
        dr = plsc.load_gather(dinv, [rv])
        dc = plsc.load_gather(dinv, [cv])
        svbuf[pl.ds(i * 16, 16)] = gv * dr * dc * vv
        return 0
    lax.fori_loop(0, (N_ADJ // 32) // 16, cbody, 0)
    pltpu.sync_copy(svbuf, svals_h.at[pl.ds(base2, N_ADJ // 32)])


_SC_CACHE = {}


def _sc_kernel(name, body, out_type, scratch_types):
    if name not in _SC_CACHE:
        _SC_CACHE[name] = pl.kernel(
            body, out_type=out_type,
            mesh=plsc.VectorSubcoreMesh(core_axis_name="c",
                                        subcore_axis_name="s"),
            scratch_types=scratch_types,
            compiler_params=pltpu.CompilerParams(needs_layout_passes=False))
    return _SC_CACHE[name]


def _svals_call(rows, cols, vals, gv):
    fn = _sc_kernel(
        'svals', _svals_body,
        jax.ShapeDtypeStruct((N_ADJ,), jnp.float32),
        _SVALS_SCRATCH)
    return fn(rows, cols, vals, gv)


_SVALS_SCRATCH = [
        pltpu.VMEM((16 * N,), jnp.float32),
        pltpu.VMEM((N,), jnp.float32),
        pltpu.VMEM((N // 16,), jnp.float32),
        pltpu.VMEM((N,), jnp.float32),
        pltpu.VMEM((N_ADJ // 16,), jnp.int32),
        pltpu.VMEM((N_ADJ // 32,), jnp.int32),
        pltpu.VMEM((N_ADJ // 16,), jnp.float32),
        pltpu.VMEM((N_ADJ // 32,), jnp.float32),
        pltpu.VMEM((16,), jnp.float32),
        pltpu.VMEM_SHARED((16, N), jnp.float32),
        pltpu.VMEM_SHARED((N,), jnp.float32),
    ]


FQ = 128


def _spmm_body(table_h, rows3_h, cols3_h, coefs_h, out_h,
               acc_sh, rbuf, idxb, cfb, gb0, gb1, gs0, gs1, *, inv):
    c = lax.axis_index("c")
    s = lax.axis_index("s")
    zero16 = jnp.zeros((16,), jnp.float32)

    pltpu.sync_copy(rows3_h.at[s], rbuf)
    pltpu.sync_copy(cols3_h.at[s], idxb)
    pltpu.sync_copy(coefs_h.at[pl.ds(s * EPT, EPT)], cfb)

    def tr(ch, _):
        for q in range(CHUNK // 16):
            sl = pl.ds(q * 16, 16)
            idxb[ch, sl] = idxb[ch, sl] * 6 + (2 * inv + c)
        return 0
    lax.fori_loop(0, NCH, tr, 0)

    def zb(r, _):
        for q in range(FQ // 16):
            gb0[r, pl.ds(q * 16, 16)] = zero16
        return 0
    lax.fori_loop(0, CHUNK, zb, 0)
    for kk in range(4):
        pltpu.sync_copy(gb0, acc_sh.at[pl.ds(s * 256 + kk * CHUNK, CHUNK)])
    plsc.subcore_barrier()

    def scale(gb, ch):
        def s1(j, _):
            cf = plsc.load_gather(
                cfb, [jnp.full((16,), ch * CHUNK + j, jnp.int32)])
            for q in range(FQ // 16):
                sl = pl.ds(q * 16, 16)
                gb[j, sl] = gb[j, sl] * cf
            return 0
        lax.fori_loop(0, CHUNK, s1, 0)

    pltpu.async_copy(table_h.at[idxb.at[0]], gb0, gs0)

    def pair(p_, _):
        ch0 = p_ * 2
        pltpu.make_async_copy(table_h.at[idxb.at[ch0]], gb0, gs0).wait()

        @pl.when(ch0 + 1 < NCH)
        def _():
            pltpu.async_copy(table_h.at[idxb.at[ch0 + 1]], gb1, gs1)
        scale(gb0, ch0)
        pltpu.sync_copy(gb0, acc_sh.at[rbuf.at[ch0]], add=True)

        ch1 = ch0 + 1
        pltpu.make_async_copy(table_h.at[idxb.at[ch1]], gb1, gs1).wait()

        @pl.when(ch1 + 1 < NCH)
        def _():
            pltpu.async_copy(table_h.at[idxb.at[ch1 + 1]], gb0, gs0)
        scale(gb1, ch1)
        pltpu.sync_copy(gb1, acc_sh.at[rbuf.at[ch1]], add=True)
        return 0
    lax.fori_loop(0, NCH // 2, pair, 0)

    plsc.subcore_barrier()
    for kk in range(4):
        row0 = s * 256 + kk * CHUNK
        pltpu.sync_copy(acc_sh.at[pl.ds(row0, CHUNK)],
                        out_h.at[pl.ds(row0, CHUNK), c])


def _spmm_call(table, rows3, cols3, coefs, inv):
    fn = _sc_kernel(
        'spmm%d' % inv, functools.partial(_spmm_body, inv=inv),
        jax.ShapeDtypeStruct((N, 2, FQ), jnp.float32),
        _SPMM_SCRATCH)
    return fn(table, rows3, cols3, coefs)


_SPMM_SCRATCH = [
        pltpu.VMEM_SHARED((N, FQ), jnp.float32),
        pltpu.VMEM((NCH, CHUNK), jnp.int32),
        pltpu.VMEM((NCH, CHUNK), jnp.int32),
        pltpu.VMEM((EPT,), jnp.float32),
        pltpu.VMEM((CHUNK, FQ), jnp.float32),
        pltpu.VMEM((CHUNK, FQ), jnp.float32),
        pltpu.SemaphoreType.DMA,
        pltpu.SemaphoreType.DMA,
    ]


def _block_body(y_ref, hprev_ref, ws, bs, wt, bt, out_ref):
    y = y_ref[...]
    hs = lax.dot_general(y, ws[...], (((2,), (0,)), ((), ())),
                         preferred_element_type=jnp.float32) + bs[...]
    hs = jnp.maximum(hs, 0.0)
    z = jnp.zeros((y.shape[0], 2, D), jnp.float32)
    hp = jnp.concatenate([z, hs], axis=1)
    dn = (((2,), (0,)), ((), ()))
    wt_v = wt[...]
    ht = (lax.dot_general(hp[:, 0:T], wt_v[0], dn,
                          preferred_element_type=jnp.float32)
          + lax.dot_general(hp[:, 1:T + 1], wt_v[1], dn,
                            preferred_element_type=jnp.float32)
          + lax.dot_general(hp[:, 2:T + 2], wt_v[2], dn,
                            preferred_element_type=jnp.float32)
          + bt[...])
    out_ref[...] = jnp.maximum(ht, 0.0) + hprev_ref[...]


def _block_call(y, hprev, ws, bs, wt, bt):
    R = 512
    return pl.pallas_call(
        _block_body,
        grid=(N // R,),
        in_specs=[
            pl.BlockSpec((R, T, D), lambda i: (i, 0, 0)),
            pl.BlockSpec((R, T, D), lambda i: (i, 0, 0)),
            _full_spec((D, D)), _full_spec((1, D)),
            _full_spec((3, D, D)), _full_spec((1, D)),
        ],
        out_specs=pl.BlockSpec((R, T, D), lambda i: (i, 0, 0)),
        out_shape=jax.ShapeDtypeStruct((N, T, D), jnp.float32),
    )(y, hprev, ws, bs.reshape(1, D), wt, bt.reshape(1, D))


def _head_body(h_ref, w1, b1, w2, b2, out_ref):
    hl = h_ref[...][:, T - 1, :]
    x = jnp.maximum(hl @ w1[...] + b1[...], 0.0)
    out_ref[...] = x @ w2[...] + b2[...]


def _head_call(h, w1, b1, w2, b2):
    R = 512
    return pl.pallas_call(
        _head_body,
        grid=(N // R,),
        in_specs=[
            pl.BlockSpec((R, T, D), lambda i: (i, 0, 0)),
            _full_spec((D, D // 2)), _full_spec((1, D // 2)),
            _full_spec((D // 2, 1)), _full_spec((1, 1)),
        ],
        out_specs=pl.BlockSpec((R, 1), lambda i: (i, 0)),
        out_shape=jax.ShapeDtypeStruct((N, 1), jnp.float32),
    )(h, w1, b1.reshape(1, D // 2), w2, b2.reshape(1, 1))


def kernel(X_edges, X_weather_edges, A_s_indices, A_s_values, params):
    p = params
    xe = X_edges.reshape(N * T, IN_FEAT)
    xw = X_weather_edges.reshape(N * T, W_DIM)
    h0 = _mlp_call(xe, xw, p).reshape(N, T, D)
    q, k = _pool_call(h0, p['Wq'], p['Wk'])

    g = jax.nn.sigmoid(p['alpha'])
    gm = (1.0 - g).reshape(1, 1).astype(jnp.float32)
    tidx, tw = _topk_call(q, k, gm)

    rows_s = A_s_indices[0].astype(jnp.int32)
    cols_s = A_s_indices[1].astype(jnp.int32)
    svals = _svals_call(rows_s, cols_s, A_s_values,
                        jnp.full((16,), g, jnp.float32))

    rows_all = jnp.concatenate(
        [jnp.repeat(jnp.arange(N, dtype=jnp.int32), TOPK), rows_s])
    cols_all = jnp.concatenate([tidx.reshape(-1), cols_s])
    coefs_all = jnp.concatenate([tw.reshape(-1), svals])
    rows3 = rows_all.reshape(16, NCH, CHUNK)
    cols3 = cols_all.reshape(16, NCH, CHUNK)

    h = h0
    for pre in ('b0_', 'b1_'):
        table = h.reshape(6 * N, FQ)
        ys = [_spmm_call(table, rows3, cols3, coefs_all, i).reshape(N, 2 * FQ)
              for i in range(3)]
        y = jnp.concatenate(ys, axis=1)
        h = _block_call(y.reshape(N, T, D), h,
                        p[pre + 'Ws'], p[pre + 'bs'],
                        p[pre + 'Wt'], p[pre + 'bt'])
    return _head_call(h, p['h_W1'], p['h_b1'], p['h_W2'], p['h_b2'])
